# Initial kernel scaffold; baseline (speedup 1.0000x reference)
import jax
import jax.numpy as jnp
from jax.experimental import pallas as pl


def kernel(context_vector, dec_hidden, x, embedding_matrix, w_s, b_s, w_c, b_c, w_i, b_i):
    raise NotImplementedError("write your pallas kernel here")


# speedup vs baseline: 1.7352x; 1.7352x over previous
import jax
import jax.numpy as jnp
from jax import lax
from jax.experimental import pallas as pl
from jax.experimental.pallas import tpu as pltpu
from jax.experimental.pallas import tpu_sc as plsc

VOCAB_N = 1000000
DIM_N = 64
HID_N = 512
B_N = 16384

_NC = 2
_NS = 16
_NW = _NC * _NS
_BPW = B_N // _NW
_NG = _BPW // 16

_SEGS = 8192
_VBLK = 8192
_NVB = (VOCAB_N + _VBLK - 1) // _VBLK


def _b0_body(tt_ref, wi_ref, out_ref):
    row = jnp.dot(wi_ref[...].T, tt_ref[...], preferred_element_type=jnp.float32)
    out_ref[...] = row.reshape(_VBLK // 128, 128)


def _tc_table_dot(table_t, w_i):
    return pl.pallas_call(
        _b0_body,
        grid=(_NVB,),
        in_specs=[
            pl.BlockSpec((DIM_N, _VBLK), lambda i: (0, i)),
            pl.BlockSpec((DIM_N, 1), lambda i: (0, 0)),
        ],
        out_specs=pl.BlockSpec((_VBLK // 128, 128), lambda i: (i, 0)),
        out_shape=jax.ShapeDtypeStruct((_SEGS, 128), jnp.float32),
    )(table_t, w_i)


def _sc_body(v, xf, out, idx_v, seg_v, lane_v, data_v, s3_v, sem):
    wid = lax.axis_index("s") * _NC + lax.axis_index("c")
    rbase = wid * _BPW
    pltpu.sync_copy(xf.at[pl.ds(rbase, _BPW)], idx_v)

    def split(g, carry):
        r = idx_v[pl.ds(g * 16, 16)]
        seg_v[pl.ds(g * 16, 16)] = r >> 7
        lane_v[pl.ds(g * 16, 16)] = r & 127
        return carry

    lax.fori_loop(0, _NG, split, 0)

    pltpu.async_copy(v.at[seg_v], data_v, sem).wait()

    lanes = lax.iota(jnp.int32, 16)

    def extract(g, carry):
        rows = g * 16 + lanes
        cols = lane_v[pl.ds(g * 16, 16)]
        s3_v[pl.ds(g * 16, 16)] = plsc.load_gather(data_v, [rows, cols])
        return carry

    lax.fori_loop(0, _NG, extract, 0)
    pltpu.sync_copy(s3_v, out.at[pl.ds(rbase, _BPW)])


def _sc_gather(v, xf):
    mesh = plsc.VectorSubcoreMesh(core_axis_name="c", subcore_axis_name="s")
    return pl.kernel(
        _sc_body,
        mesh=mesh,
        out_type=jax.ShapeDtypeStruct((B_N,), jnp.float32),
        scratch_types=[
            pltpu.VMEM((_BPW,), jnp.int32),
            pltpu.VMEM((_BPW,), jnp.int32),
            pltpu.VMEM((_BPW,), jnp.int32),
            pltpu.VMEM((_BPW, 128), jnp.float32),
            pltpu.VMEM((_BPW,), jnp.float32),
            pltpu.SemaphoreType.DMA,
        ],
        compiler_params=pltpu.CompilerParams(needs_layout_passes=False),
    )(v, xf)


_BLK = 2048


def _b1_body(dh_ref, cv_ref, ws_ref, wc_ref, out_ref):
    acc = jnp.dot(dh_ref[...], ws_ref[...], preferred_element_type=jnp.float32)
    acc = acc + jnp.dot(cv_ref[...], wc_ref[...], preferred_element_type=jnp.float32)
    out_ref[...] = acc


def _tc_dense(dec_hidden, context_vector, w_s, w_c):
    return pl.pallas_call(
        _b1_body,
        grid=(B_N // _BLK,),
        in_specs=[
            pl.BlockSpec((_BLK, HID_N), lambda i: (i, 0)),
            pl.BlockSpec((_BLK, HID_N), lambda i: (i, 0)),
            pl.BlockSpec((HID_N, 1), lambda i: (0, 0)),
            pl.BlockSpec((HID_N, 1), lambda i: (0, 0)),
        ],
        out_specs=pl.BlockSpec((_BLK, 1), lambda i: (i, 0)),
        out_shape=jax.ShapeDtypeStruct((B_N, 1), jnp.float32),
    )(dec_hidden, context_vector, w_s, w_c)


def _b2_body(s12_ref, s3_ref, bias_ref, out_ref):
    z = s12_ref[...] + s3_ref[...].reshape(_BLK, 1) + bias_ref[0, 0]
    out_ref[...] = 1.0 / (1.0 + jnp.exp(-z))


def _tc_combine(s12, s3, bias):
    return pl.pallas_call(
        _b2_body,
        grid=(B_N // _BLK,),
        in_specs=[
            pl.BlockSpec((_BLK, 1), lambda i: (i, 0)),
            pl.BlockSpec((_BLK,), lambda i: (i,)),
            pl.BlockSpec((1, 1), lambda i: (0, 0), memory_space=pltpu.SMEM),
        ],
        out_specs=pl.BlockSpec((_BLK, 1), lambda i: (i, 0)),
        out_shape=jax.ShapeDtypeStruct((B_N, 1), jnp.float32),
    )(s12, s3, bias)


def kernel(context_vector, dec_hidden, x, embedding_matrix, w_s, b_s, w_c, b_c, w_i, b_i):
    xf = x.reshape(B_N).astype(jnp.int32)
    table_t = embedding_matrix.T
    v = _tc_table_dot(table_t, w_i)
    s3 = _sc_gather(v, xf)
    s12 = _tc_dense(dec_hidden, context_vector, w_s, w_c)
    bias = (b_s + b_c + b_i).reshape(1, 1)
    return _tc_combine(s12, s3, bias)


# speedup vs baseline: 2.3685x; 1.3650x over previous
import jax
import jax.numpy as jnp
from jax import lax
from jax.experimental import pallas as pl
from jax.experimental.pallas import tpu as pltpu
from jax.experimental.pallas import tpu_sc as plsc

VOCAB_N = 1000000
DIM_N = 64
HID_N = 512
B_N = 16384

_NC = 2
_NS = 16
_NW = _NC * _NS
_BPW = B_N // _NW
_NG = _BPW // 16

_SEGS = 8192
_VBLK = 32768
_NVB = (VOCAB_N + _VBLK - 1) // _VBLK


def _b0_body(tt_ref, wi_ref, out_ref):
    row = jnp.dot(wi_ref[...].T, tt_ref[...], preferred_element_type=jnp.float32)
    out_ref[...] = row.reshape(_VBLK // 128, 128)


def _tc_table_dot(table_t, w_i):
    return pl.pallas_call(
        _b0_body,
        grid=(_NVB,),
        in_specs=[
            pl.BlockSpec((DIM_N, _VBLK), lambda i: (0, i)),
            pl.BlockSpec((DIM_N, 1), lambda i: (0, 0)),
        ],
        out_specs=pl.BlockSpec((_VBLK // 128, 128), lambda i: (i, 0)),
        out_shape=jax.ShapeDtypeStruct((_SEGS, 128), jnp.float32),
    )(table_t, w_i)


def _sc_body(v, xf, out, idx_v, seg_v, lane_v, data_v, s3_v, sem):
    wid = lax.axis_index("s") * _NC + lax.axis_index("c")
    rbase = wid * _BPW
    pltpu.sync_copy(xf.at[pl.ds(rbase, _BPW)], idx_v)

    def split(g, carry):
        r = idx_v[pl.ds(g * 16, 16)]
        seg_v[pl.ds(g * 16, 16)] = r >> 7
        lane_v[pl.ds(g * 16, 16)] = r & 127
        return carry

    lax.fori_loop(0, _NG, split, 0)

    pltpu.async_copy(v.at[seg_v], data_v, sem).wait()

    lanes = lax.iota(jnp.int32, 16)

    def extract(g, carry):
        rows = g * 16 + lanes
        cols = lane_v[pl.ds(g * 16, 16)]
        s3_v[pl.ds(g * 16, 16)] = plsc.load_gather(data_v, [rows, cols])
        return carry

    lax.fori_loop(0, _NG, extract, 0)
    pltpu.sync_copy(s3_v, out.at[pl.ds(rbase, _BPW)])


def _sc_gather(v, xf):
    mesh = plsc.VectorSubcoreMesh(core_axis_name="c", subcore_axis_name="s")
    return pl.kernel(
        _sc_body,
        mesh=mesh,
        out_type=jax.ShapeDtypeStruct((B_N,), jnp.float32),
        scratch_types=[
            pltpu.VMEM((_BPW,), jnp.int32),
            pltpu.VMEM((_BPW,), jnp.int32),
            pltpu.VMEM((_BPW,), jnp.int32),
            pltpu.VMEM((_BPW, 128), jnp.float32),
            pltpu.VMEM((_BPW,), jnp.float32),
            pltpu.SemaphoreType.DMA,
        ],
        compiler_params=pltpu.CompilerParams(needs_layout_passes=False),
    )(v, xf)


_BLK = 4096


def _b1_body(dh_ref, cv_ref, ws_ref, wc_ref, out_ref):
    acc = jnp.dot(dh_ref[...], ws_ref[...], preferred_element_type=jnp.float32)
    acc = acc + jnp.dot(cv_ref[...], wc_ref[...], preferred_element_type=jnp.float32)
    out_ref[...] = acc


def _tc_dense(dec_hidden, context_vector, w_s, w_c):
    return pl.pallas_call(
        _b1_body,
        grid=(B_N // _BLK,),
        in_specs=[
            pl.BlockSpec((_BLK, HID_N), lambda i: (i, 0)),
            pl.BlockSpec((_BLK, HID_N), lambda i: (i, 0)),
            pl.BlockSpec((HID_N, 1), lambda i: (0, 0)),
            pl.BlockSpec((HID_N, 1), lambda i: (0, 0)),
        ],
        out_specs=pl.BlockSpec((_BLK, 1), lambda i: (i, 0)),
        out_shape=jax.ShapeDtypeStruct((B_N, 1), jnp.float32),
    )(dec_hidden, context_vector, w_s, w_c)


def _b2_body(s12_ref, s3_ref, bias_ref, out_ref):
    z = s12_ref[...] + s3_ref[...].reshape(_BLK, 1) + bias_ref[0, 0]
    out_ref[...] = 1.0 / (1.0 + jnp.exp(-z))


def _tc_combine(s12, s3, bias):
    return pl.pallas_call(
        _b2_body,
        grid=(B_N // _BLK,),
        in_specs=[
            pl.BlockSpec((_BLK, 1), lambda i: (i, 0)),
            pl.BlockSpec((_BLK,), lambda i: (i,)),
            pl.BlockSpec((1, 1), lambda i: (0, 0), memory_space=pltpu.SMEM),
        ],
        out_specs=pl.BlockSpec((_BLK, 1), lambda i: (i, 0)),
        out_shape=jax.ShapeDtypeStruct((B_N, 1), jnp.float32),
    )(s12, s3, bias)


def kernel(context_vector, dec_hidden, x, embedding_matrix, w_s, b_s, w_c, b_c, w_i, b_i):
    xf = x.reshape(B_N).astype(jnp.int32)
    table_t = embedding_matrix.T
    v = _tc_table_dot(table_t, w_i)
    s3 = _sc_gather(v, xf)
    s12 = _tc_dense(dec_hidden, context_vector, w_s, w_c)
    bias = (b_s + b_c + b_i).reshape(1, 1)
    return _tc_combine(s12, s3, bias)
